# Initial kernel scaffold; baseline (speedup 1.0000x reference)
import jax
import jax.numpy as jnp
from jax.experimental import pallas as pl


def kernel(hidden_states, W, b):
    raise NotImplementedError("write your pallas kernel here")


# speedup vs baseline: 1.3880x; 1.3880x over previous
import jax
import jax.numpy as jnp
from jax.experimental import pallas as pl

T = 32768
H = 768
E = 64
TOP_K = 2
BT = 512


def _router_block(x_ref, w_ref, b_ref, logits_ref, aff_ref, idx_ref):
    x = x_ref[...]
    w = w_ref[...]
    b = b_ref[...]
    logits = jax.lax.dot_general(
        x, w, (((1,), (0,)), ((), ())),
        preferred_element_type=jnp.float32) + b
    logits_ref[...] = logits

    m = jnp.max(logits, axis=1, keepdims=True)
    ex = jnp.exp(logits - m)
    aff = ex / jnp.sum(ex, axis=1, keepdims=True)
    aff_ref[...] = aff

    iota = jax.lax.broadcasted_iota(jnp.int32, (BT, E), 1)
    big = jnp.int32(E)
    top1 = jnp.max(aff, axis=1, keepdims=True)
    idx1 = jnp.min(jnp.where(aff == top1, iota, big), axis=1, keepdims=True)
    masked = jnp.where(iota == idx1, -jnp.inf, aff)
    top2 = jnp.max(masked, axis=1, keepdims=True)
    idx2 = jnp.min(jnp.where(masked == top2, iota, big), axis=1, keepdims=True)
    idx_ref[...] = jnp.concatenate([idx1, idx2], axis=1)


def kernel(hidden_states, W, b):
    b2 = b.reshape(1, E)
    grid = (T // BT,)
    logits, aff, idx = pl.pallas_call(
        _router_block,
        grid=grid,
        in_specs=[
            pl.BlockSpec((BT, H), lambda i: (i, 0)),
            pl.BlockSpec((H, E), lambda i: (0, 0)),
            pl.BlockSpec((1, E), lambda i: (0, 0)),
        ],
        out_specs=[
            pl.BlockSpec((BT, E), lambda i: (i, 0)),
            pl.BlockSpec((BT, E), lambda i: (i, 0)),
            pl.BlockSpec((BT, TOP_K), lambda i: (i, 0)),
        ],
        out_shape=[
            jax.ShapeDtypeStruct((T, E), jnp.float32),
            jax.ShapeDtypeStruct((T, E), jnp.float32),
            jax.ShapeDtypeStruct((T, TOP_K), jnp.int32),
        ],
    )(hidden_states, W, b2)
    return (logits, aff, idx)


# speedup vs baseline: 1.6815x; 1.2114x over previous
import jax
import jax.numpy as jnp
from jax.experimental import pallas as pl

T = 32768
H = 768
E = 64
TOP_K = 2
BT = 1024


def _router_block(x_ref, w_ref, b_ref, logits_ref, aff_ref, idx_ref):
    x = x_ref[...]
    w = w_ref[...]
    b = b_ref[...]
    logits = jax.lax.dot_general(
        x, w, (((1,), (0,)), ((), ())),
        preferred_element_type=jnp.float32) + b
    logits_ref[...] = logits

    m = jnp.max(logits, axis=1, keepdims=True)
    ex = jnp.exp(logits - m)
    aff = ex / jnp.sum(ex, axis=1, keepdims=True)
    aff_ref[...] = aff

    iota = jax.lax.broadcasted_iota(jnp.int32, (BT, E), 1)
    big = jnp.int32(E)
    top1 = jnp.max(aff, axis=1, keepdims=True)
    idx1 = jnp.min(jnp.where(aff == top1, iota, big), axis=1, keepdims=True)
    masked = jnp.where(iota == idx1, -jnp.inf, aff)
    top2 = jnp.max(masked, axis=1, keepdims=True)
    idx2 = jnp.min(jnp.where(masked == top2, iota, big), axis=1, keepdims=True)
    idx_ref[...] = jnp.concatenate([idx1, idx2], axis=1)


def kernel(hidden_states, W, b):
    b2 = b.reshape(1, E)
    grid = (T // BT,)
    logits, aff, idx = pl.pallas_call(
        _router_block,
        grid=grid,
        in_specs=[
            pl.BlockSpec((BT, H), lambda i: (i, 0)),
            pl.BlockSpec((H, E), lambda i: (0, 0)),
            pl.BlockSpec((1, E), lambda i: (0, 0)),
        ],
        out_specs=[
            pl.BlockSpec((BT, E), lambda i: (i, 0)),
            pl.BlockSpec((BT, E), lambda i: (i, 0)),
            pl.BlockSpec((BT, TOP_K), lambda i: (i, 0)),
        ],
        out_shape=[
            jax.ShapeDtypeStruct((T, E), jnp.float32),
            jax.ShapeDtypeStruct((T, E), jnp.float32),
            jax.ShapeDtypeStruct((T, TOP_K), jnp.int32),
        ],
    )(hidden_states, W, b2)
    return (logits, aff, idx)


# speedup vs baseline: 1.8676x; 1.1107x over previous
import jax
import jax.numpy as jnp
from jax.experimental import pallas as pl

T = 32768
H = 768
E = 64
TOP_K = 2
BT = 2048


def _router_block(x_ref, w_ref, b_ref, logits_ref, aff_ref, idx_ref):
    x = x_ref[...]
    w = w_ref[...]
    b = b_ref[...]
    logits = jax.lax.dot_general(
        x, w, (((1,), (0,)), ((), ())),
        preferred_element_type=jnp.float32) + b
    logits_ref[...] = logits

    m = jnp.max(logits, axis=1, keepdims=True)
    ex = jnp.exp(logits - m)
    aff = ex / jnp.sum(ex, axis=1, keepdims=True)
    aff_ref[...] = aff

    iota = jax.lax.broadcasted_iota(jnp.int32, (BT, E), 1)
    big = jnp.int32(E)
    top1 = jnp.max(aff, axis=1, keepdims=True)
    idx1 = jnp.min(jnp.where(aff == top1, iota, big), axis=1, keepdims=True)
    masked = jnp.where(iota == idx1, -jnp.inf, aff)
    top2 = jnp.max(masked, axis=1, keepdims=True)
    idx2 = jnp.min(jnp.where(masked == top2, iota, big), axis=1, keepdims=True)
    idx_ref[...] = jnp.concatenate([idx1, idx2], axis=1)


def kernel(hidden_states, W, b):
    b2 = b.reshape(1, E)
    grid = (T // BT,)
    logits, aff, idx = pl.pallas_call(
        _router_block,
        grid=grid,
        in_specs=[
            pl.BlockSpec((BT, H), lambda i: (i, 0)),
            pl.BlockSpec((H, E), lambda i: (0, 0)),
            pl.BlockSpec((1, E), lambda i: (0, 0)),
        ],
        out_specs=[
            pl.BlockSpec((BT, E), lambda i: (i, 0)),
            pl.BlockSpec((BT, E), lambda i: (i, 0)),
            pl.BlockSpec((BT, TOP_K), lambda i: (i, 0)),
        ],
        out_shape=[
            jax.ShapeDtypeStruct((T, E), jnp.float32),
            jax.ShapeDtypeStruct((T, E), jnp.float32),
            jax.ShapeDtypeStruct((T, TOP_K), jnp.int32),
        ],
    )(hidden_states, W, b2)
    return (logits, aff, idx)


# speedup vs baseline: 1.9488x; 1.0435x over previous
import jax
import jax.numpy as jnp
from jax.experimental import pallas as pl

T = 32768
H = 768
E = 64
TOP_K = 2
BT = 4096


def _router_block(x_ref, w_ref, b_ref, logits_ref, aff_ref, idx_ref):
    x = x_ref[...]
    w = w_ref[...]
    b = b_ref[...]
    logits = jax.lax.dot_general(
        x, w, (((1,), (0,)), ((), ())),
        preferred_element_type=jnp.float32) + b
    logits_ref[...] = logits

    m = jnp.max(logits, axis=1, keepdims=True)
    ex = jnp.exp(logits - m)
    aff = ex / jnp.sum(ex, axis=1, keepdims=True)
    aff_ref[...] = aff

    iota = jax.lax.broadcasted_iota(jnp.int32, (BT, E), 1)
    big = jnp.int32(E)
    top1 = jnp.max(aff, axis=1, keepdims=True)
    idx1 = jnp.min(jnp.where(aff == top1, iota, big), axis=1, keepdims=True)
    masked = jnp.where(iota == idx1, -jnp.inf, aff)
    top2 = jnp.max(masked, axis=1, keepdims=True)
    idx2 = jnp.min(jnp.where(masked == top2, iota, big), axis=1, keepdims=True)
    idx_ref[...] = jnp.concatenate([idx1, idx2], axis=1)


def kernel(hidden_states, W, b):
    b2 = b.reshape(1, E)
    grid = (T // BT,)
    logits, aff, idx = pl.pallas_call(
        _router_block,
        grid=grid,
        in_specs=[
            pl.BlockSpec((BT, H), lambda i: (i, 0)),
            pl.BlockSpec((H, E), lambda i: (0, 0)),
            pl.BlockSpec((1, E), lambda i: (0, 0)),
        ],
        out_specs=[
            pl.BlockSpec((BT, E), lambda i: (i, 0)),
            pl.BlockSpec((BT, E), lambda i: (i, 0)),
            pl.BlockSpec((BT, TOP_K), lambda i: (i, 0)),
        ],
        out_shape=[
            jax.ShapeDtypeStruct((T, E), jnp.float32),
            jax.ShapeDtypeStruct((T, E), jnp.float32),
            jax.ShapeDtypeStruct((T, TOP_K), jnp.int32),
        ],
    )(hidden_states, W, b2)
    return (logits, aff, idx)
